# Initial kernel scaffold; baseline (speedup 1.0000x reference)
import jax
import jax.numpy as jnp
from jax.experimental import pallas as pl


def kernel(inputs, graph, W_in, b_in, W_conv0, b_conv0, W_conv1, b_conv1, W_gate, b_gate, W_cls, b_cls):
    raise NotImplementedError("write your pallas kernel here")


# speedup vs baseline: 10.1637x; 10.1637x over previous
import functools

import jax
import jax.numpy as jnp
from jax import lax
from jax.experimental import pallas as pl
from jax.experimental.pallas import tpu as pltpu
from jax.experimental.pallas import tpu_sc as plsc

N = 10000
NP = 10240
E = 320000
D = 128
D_OUT = 64

NC = 2
NS = 16
NW = NC * NS
EPW = E // NW
K = 80
NCHUNK = EPW // K
RPS = NP // NS
ZR = 128

R = 1000


def _mesh():
    return plsc.VectorSubcoreMesh(core_axis_name="c", subcore_axis_name="s")


def _sc_degrees(src, dst, ones_h, zeros_h):
    @functools.partial(
        pl.kernel,
        mesh=_mesh(),
        out_type=jax.ShapeDtypeStruct((NC, 2, NP), jnp.float32),
        scratch_types=[
            pltpu.VMEM((K,), jnp.int32),
            pltpu.VMEM((K,), jnp.int32),
            pltpu.VMEM((K,), jnp.float32),
            pltpu.VMEM_SHARED((NP,), jnp.float32),
            pltpu.VMEM_SHARED((NP,), jnp.float32),
        ],
    )
    def deg_kernel(src_hbm, dst_hbm, ones_hbm, zeros_hbm, out_hbm,
                   src_v, dst_v, ones_v, acc_o, acc_i):
        c = lax.axis_index("c")
        s = lax.axis_index("s")
        wid = s * NC + c
        pltpu.sync_copy(zeros_hbm, acc_o.at[pl.ds(s * RPS, RPS)])
        pltpu.sync_copy(zeros_hbm, acc_i.at[pl.ds(s * RPS, RPS)])
        pltpu.sync_copy(ones_hbm, ones_v)
        plsc.subcore_barrier()
        base = wid * EPW

        def step(i, carry):
            off = base + i * K
            pltpu.sync_copy(src_hbm.at[pl.ds(off, K)], src_v)
            pltpu.sync_copy(dst_hbm.at[pl.ds(off, K)], dst_v)
            pltpu.sync_copy(ones_v, acc_o.at[src_v], add=True)
            pltpu.sync_copy(ones_v, acc_i.at[dst_v], add=True)
            return carry

        lax.fori_loop(0, NCHUNK, step, 0)
        plsc.subcore_barrier()
        pltpu.sync_copy(acc_o.at[pl.ds(s * RPS, RPS)],
                        out_hbm.at[c, 0, pl.ds(s * RPS, RPS)])
        pltpu.sync_copy(acc_i.at[pl.ds(s * RPS, RPS)],
                        out_hbm.at[c, 1, pl.ds(s * RPS, RPS)])

    return deg_kernel(src, dst, ones_h, zeros_h)


def _sc_scatter(x, src, dst, zeros_h):
    @functools.partial(
        pl.kernel,
        mesh=_mesh(),
        out_type=jax.ShapeDtypeStruct((NC, NP, D), jnp.float32),
        scratch_types=[
            pltpu.VMEM((K,), jnp.int32),
            pltpu.VMEM((K,), jnp.int32),
            pltpu.VMEM((K, D), jnp.float32),
            pltpu.VMEM_SHARED((NP, D), jnp.float32),
            pltpu.SemaphoreType.DMA,
        ],
    )
    def scat_kernel(x_hbm, src_hbm, dst_hbm, zeros_hbm, out_hbm,
                    src_v, dst_v, rows_v, acc, sem):
        c = lax.axis_index("c")
        s = lax.axis_index("s")
        wid = s * NC + c
        for z in range(RPS // ZR):
            pltpu.sync_copy(zeros_hbm, acc.at[pl.ds(s * RPS + z * ZR, ZR)])
        plsc.subcore_barrier()
        base = wid * EPW

        def step(i, carry):
            off = base + i * K
            pltpu.sync_copy(src_hbm.at[pl.ds(off, K)], src_v)
            pltpu.sync_copy(dst_hbm.at[pl.ds(off, K)], dst_v)
            pltpu.async_copy(x_hbm.at[src_v], rows_v, sem).wait()
            pltpu.sync_copy(rows_v, acc.at[dst_v], add=True)
            return carry

        lax.fori_loop(0, NCHUNK, step, 0)
        plsc.subcore_barrier()
        pltpu.sync_copy(acc.at[pl.ds(s * RPS, RPS)],
                        out_hbm.at[c, pl.ds(s * RPS, RPS)])

    return scat_kernel(x, src, dst, zeros_h)


def _scales(deg_ref):
    s_out = lax.rsqrt(jnp.maximum(deg_ref[0, 0] + deg_ref[1, 0], 1.0))
    s_in = lax.rsqrt(jnp.maximum(deg_ref[0, 1] + deg_ref[1, 1], 1.0))
    return s_out, s_in


def _dot(a, b):
    return jnp.dot(a, b, preferred_element_type=jnp.float32)


def _tc_in_body(deg_ref, x_ref, win_ref, bin_ref, wc_ref, h_ref, x0_ref):
    s_out, _ = _scales(deg_ref)
    h = jnp.maximum(_dot(x_ref[...], win_ref[...]) + bin_ref[...], 0.0)
    h_ref[...] = h
    x0_ref[...] = _dot(h * s_out, wc_ref[...])


def _tc_gate_body(deg_ref, h_ref, p_ref, bc_ref, wg_ref, bg_ref, wc_ref,
                  h1_ref, x1_ref):
    s_out, s_in = _scales(deg_ref)
    cell = (p_ref[0] + p_ref[1]) * s_in + bc_ref[...]
    h = h_ref[...]
    wg = wg_ref[...]
    cc = jax.nn.sigmoid(_dot(h, wg[0:D]) + _dot(cell, wg[D:2 * D])
                        + bg_ref[...])
    h1_ref[...] = h * cc[:, 0:1] + cell * cc[:, 1:2]
    x1_ref[...] = _dot(cell * cc[:, 3:4] * s_out, wc_ref[...])


def _tc_out_body(deg_ref, h_ref, p_ref, bc_ref, wg_ref, bg_ref, wcls_ref,
                 bcls_ref, out_ref):
    _, s_in = _scales(deg_ref)
    cell = (p_ref[0] + p_ref[1]) * s_in + bc_ref[...]
    h = h_ref[...]
    wg = wg_ref[...]
    cc = jax.nn.sigmoid(_dot(h, wg[0:D]) + _dot(cell, wg[D:2 * D])
                        + bg_ref[...])
    h2 = h * cc[:, 0:1] + cell * cc[:, 1:2]
    out_ref[...] = _dot(h2, wcls_ref[...]) + bcls_ref[...]


def _spec_deg():
    return pl.BlockSpec((NC, 2, R, 1), lambda i: (0, 0, i, 0))


def _spec_rows():
    return pl.BlockSpec((R, D), lambda i: (i, 0))


def _spec_full(shape):
    nd = len(shape)
    return pl.BlockSpec(shape, lambda i: (0,) * nd)


def _tc_in(degp, x, w_in, b_in, w_conv):
    return pl.pallas_call(
        _tc_in_body,
        grid=(N // R,),
        in_specs=[
            _spec_deg(), _spec_rows(),
            _spec_full((D, D)), _spec_full((1, D)), _spec_full((D, D)),
        ],
        out_specs=[_spec_rows(), _spec_rows()],
        out_shape=[jax.ShapeDtypeStruct((N, D), jnp.float32),
                   jax.ShapeDtypeStruct((N, D), jnp.float32)],
    )(degp, x, w_in, b_in, w_conv)


def _tc_gate(degp, h, p, b_conv, w_gate, b_gate, w_conv_next):
    return pl.pallas_call(
        _tc_gate_body,
        grid=(N // R,),
        in_specs=[
            _spec_deg(), _spec_rows(),
            pl.BlockSpec((NC, R, D), lambda i: (0, i, 0)),
            _spec_full((1, D)), _spec_full((2 * D, 4)), _spec_full((1, 4)),
            _spec_full((D, D)),
        ],
        out_specs=[_spec_rows(), _spec_rows()],
        out_shape=[jax.ShapeDtypeStruct((N, D), jnp.float32),
                   jax.ShapeDtypeStruct((N, D), jnp.float32)],
    )(degp, h, p, b_conv, w_gate, b_gate, w_conv_next)


def _tc_out(degp, h, p, b_conv, w_gate, b_gate, w_cls, b_cls):
    return pl.pallas_call(
        _tc_out_body,
        grid=(N // R,),
        in_specs=[
            _spec_deg(), _spec_rows(),
            pl.BlockSpec((NC, R, D), lambda i: (0, i, 0)),
            _spec_full((1, D)), _spec_full((2 * D, 4)), _spec_full((1, 4)),
            _spec_full((D, D_OUT)), _spec_full((1, D_OUT)),
        ],
        out_specs=pl.BlockSpec((R, D_OUT), lambda i: (i, 0)),
        out_shape=jax.ShapeDtypeStruct((N, D_OUT), jnp.float32),
    )(degp, h, p, b_conv, w_gate, b_gate, w_cls, b_cls)


def kernel(inputs, graph, W_in, b_in, W_conv0, b_conv0, W_conv1, b_conv1,
           W_gate, b_gate, W_cls, b_cls):
    src = graph[0]
    dst = graph[1]
    ones_h = jnp.ones((K,), jnp.float32)
    zeros_deg = jnp.zeros((RPS,), jnp.float32)
    zeros_row = jnp.zeros((ZR, D), jnp.float32)

    degp = _sc_degrees(src, dst, ones_h, zeros_deg).reshape(NC, 2, NP, 1)

    h, x0 = _tc_in(degp, inputs, W_in, b_in.reshape(1, D), W_conv0)
    p0 = _sc_scatter(x0, src, dst, zeros_row)
    h1, x1 = _tc_gate(degp, h, p0, b_conv0.reshape(1, D), W_gate,
                      b_gate.reshape(1, 4), W_conv1)
    p1 = _sc_scatter(x1, src, dst, zeros_row)
    out = _tc_out(degp, h1, p1, b_conv1.reshape(1, D), W_gate,
                  b_gate.reshape(1, 4), W_cls, b_cls.reshape(1, D_OUT))
    return out
